# Initial kernel scaffold; baseline (speedup 1.0000x reference)
import functools

import jax
import jax.numpy as jnp
from jax import lax
from jax.experimental import pallas as pl
from jax.experimental.pallas import tpu as pltpu
from jax.experimental.pallas import tpu_sc as plsc

_NC = 2
_NS = 16
_NW = _NC * _NS
_CHUNK = 128


def _make_gather(B: int, D: int, n_chunks: int):
  mesh = plsc.VectorSubcoreMesh(core_axis_name="c", subcore_axis_name="s")

  @functools.partial(
      pl.kernel,
      out_type=jax.ShapeDtypeStruct((B, D), jnp.float32),
      mesh=mesh,
      scratch_types=[
          pltpu.VMEM((n_chunks, _CHUNK), jnp.int32),
          pltpu.VMEM((_CHUNK, D), jnp.float32),
          pltpu.SemaphoreType.DMA,
      ],
  )
  def gather_kernel(table_hbm, idx_hbm, out_hbm, idx_v, rows_v, sem):
    wid = lax.axis_index("s") * _NC + lax.axis_index("c")
    base = wid * (n_chunks * _CHUNK)
    pltpu.sync_copy(idx_hbm.at[wid], idx_v)

    def body(j, carry):
      pltpu.async_copy(table_hbm.at[idx_v.at[j]], rows_v, sem).wait()
      pltpu.sync_copy(rows_v, out_hbm.at[pl.ds(base + j * _CHUNK, _CHUNK)])
      return carry

    lax.fori_loop(0, n_chunks, body, 0)

  return gather_kernel


def kernel(x, weight):
  BATCH, HIST = x.shape
  V, D = weight.shape
  B = BATCH * HIST
  assert B % (_NW * _CHUNK) == 0
  n_chunks = B // (_NW * _CHUNK)
  idx = x.reshape(_NW, n_chunks, _CHUNK).astype(jnp.int32)
  out = _make_gather(B, D, n_chunks)(weight, idx)
  return out.reshape(BATCH, HIST, D)


# speedup vs baseline: 1.6830x; 1.6830x over previous
import functools

import jax
import jax.numpy as jnp
from jax import lax
from jax.experimental import pallas as pl
from jax.experimental.pallas import tpu as pltpu
from jax.experimental.pallas import tpu_sc as plsc

_NC = 2
_NS = 16
_NW = _NC * _NS
_CHUNK = 128


def _make_gather(B: int, D: int, n_chunks: int):
  mesh = plsc.VectorSubcoreMesh(core_axis_name="c", subcore_axis_name="s")

  @functools.partial(
      pl.kernel,
      out_type=jax.ShapeDtypeStruct((B, D), jnp.float32),
      mesh=mesh,
      scratch_types=[
          pltpu.VMEM((n_chunks, _CHUNK), jnp.int32),
          pltpu.VMEM((_CHUNK, D), jnp.float32),
          pltpu.SemaphoreType.DMA,
      ],
      compiler_params=pltpu.CompilerParams(use_tc_tiling_on_sc=False),
  )
  def gather_kernel(table_hbm, idx_hbm, out_hbm, idx_v, rows_v, sem):
    wid = lax.axis_index("s") * _NC + lax.axis_index("c")
    base = wid * (n_chunks * _CHUNK)
    pltpu.sync_copy(idx_hbm.at[wid], idx_v)

    def body(j, carry):
      pltpu.async_copy(table_hbm.at[idx_v.at[j]], rows_v, sem).wait()
      pltpu.sync_copy(rows_v, out_hbm.at[pl.ds(base + j * _CHUNK, _CHUNK)])
      return carry

    lax.fori_loop(0, n_chunks, body, 0)

  return gather_kernel


def kernel(x, weight):
  BATCH, HIST = x.shape
  V, D = weight.shape
  B = BATCH * HIST
  assert B % (_NW * _CHUNK) == 0
  n_chunks = B // (_NW * _CHUNK)
  idx = x.reshape(_NW, n_chunks, _CHUNK).astype(jnp.int32)
  out = _make_gather(B, D, n_chunks)(weight, idx)
  return out.reshape(BATCH, HIST, D)


# speedup vs baseline: 1.8782x; 1.1160x over previous
import functools

import jax
import jax.numpy as jnp
from jax import lax
from jax.experimental import pallas as pl
from jax.experimental.pallas import tpu as pltpu
from jax.experimental.pallas import tpu_sc as plsc

_NC = 2
_NS = 16
_NW = _NC * _NS
_CHUNK = 128
_NBUF = 8
_LOOKAHEAD = 4


def _make_gather(B: int, D: int, n_chunks: int):
  mesh = plsc.VectorSubcoreMesh(core_axis_name="c", subcore_axis_name="s")

  @functools.partial(
      pl.kernel,
      out_type=jax.ShapeDtypeStruct((B, D), jnp.float32),
      mesh=mesh,
      scratch_types=[
          pltpu.VMEM((n_chunks, _CHUNK), jnp.int32),
          pltpu.VMEM((_NBUF, _CHUNK, D), jnp.float32),
          pltpu.SemaphoreType.DMA,
          pltpu.SemaphoreType.DMA,
      ],
      compiler_params=pltpu.CompilerParams(use_tc_tiling_on_sc=False),
  )
  def gather_kernel(table_hbm, idx_hbm, out_hbm, idx_v, rows, gsem, wsem):
    wid = lax.axis_index("s") * _NC + lax.axis_index("c")
    base = wid * (n_chunks * _CHUNK)
    pltpu.sync_copy(idx_hbm.at[wid], idx_v)

    def g(j, b):
      pltpu.async_copy(table_hbm.at[idx_v.at[j]], rows.at[b], gsem)

    def wg(b):
      pltpu.make_async_copy(
          table_hbm.at[idx_v.at[0]], rows.at[b], gsem).wait()

    def w(j, b):
      pltpu.async_copy(
          rows.at[b], out_hbm.at[pl.ds(base + j * _CHUNK, _CHUNK)], wsem)

    def ww(b):
      pltpu.make_async_copy(
          rows.at[b], out_hbm.at[pl.ds(base, _CHUNK)], wsem).wait()

    LA, NB = _LOOKAHEAD, _NBUF
    n_groups = n_chunks // NB

    for b in range(LA):
      g(b, b)

    for b in range(NB):
      wg(b)
      w(b, b)
      bn = (b + LA) % NB
      if b >= LA:
        ww(bn)
      g(b + LA, bn)

    def body(k, carry):
      j0 = k * NB
      for b in range(NB):
        wg(b)
        w(j0 + b, b)
        bn = (b + LA) % NB
        ww(bn)
        g(j0 + b + LA, bn)
      return carry

    lax.fori_loop(1, n_groups - 1, body, 0)

    j0 = (n_groups - 1) * NB
    for b in range(NB):
      wg(b)
      w(j0 + b, b)
      if b < NB - LA:
        bn = (b + LA) % NB
        ww(bn)
        g(j0 + b + LA, bn)

    for b in range(NB):
      ww(b)

  return gather_kernel


def kernel(x, weight):
  BATCH, HIST = x.shape
  V, D = weight.shape
  B = BATCH * HIST
  assert B % (_NW * _CHUNK) == 0
  n_chunks = B // (_NW * _CHUNK)
  idx = x.reshape(_NW, n_chunks, _CHUNK).astype(jnp.int32)
  out = _make_gather(B, D, n_chunks)(weight, idx)
  return out.reshape(BATCH, HIST, D)
